# Initial kernel scaffold; baseline (speedup 1.0000x reference)
import jax
import jax.numpy as jnp
from jax.experimental import pallas as pl


def kernel(x, pos_table):
    raise NotImplementedError("write your pallas kernel here")


# speedup vs baseline: 1.4852x; 1.4852x over previous
import jax
import jax.numpy as jnp
from jax.experimental import pallas as pl


def _body(x_ref, p_ref, o_ref):
    o_ref[...] = x_ref[...] + p_ref[...]


def kernel(x, pos_table):
    B, S, D = x.shape
    BLK = 512
    return pl.pallas_call(
        _body,
        grid=(S // BLK, B),
        in_specs=[
            pl.BlockSpec((1, BLK, D), lambda s, b: (b, s, 0)),
            pl.BlockSpec((BLK, D), lambda s, b: (s, 0)),
        ],
        out_specs=pl.BlockSpec((1, BLK, D), lambda s, b: (b, s, 0)),
        out_shape=jax.ShapeDtypeStruct((B, S, D), x.dtype),
    )(x, pos_table)


# speedup vs baseline: 1.6643x; 1.1206x over previous
import jax
import jax.numpy as jnp
from jax.experimental import pallas as pl


def _body(x_ref, p_ref, o_ref):
    o_ref[...] = x_ref[...] + p_ref[...]


def kernel(x, pos_table):
    B, S, D = x.shape
    BLK = 1024
    return pl.pallas_call(
        _body,
        grid=(S // BLK, B),
        in_specs=[
            pl.BlockSpec((1, BLK, D), lambda s, b: (b, s, 0)),
            pl.BlockSpec((BLK, D), lambda s, b: (s, 0)),
        ],
        out_specs=pl.BlockSpec((1, BLK, D), lambda s, b: (b, s, 0)),
        out_shape=jax.ShapeDtypeStruct((B, S, D), x.dtype),
    )(x, pos_table)


# speedup vs baseline: 1.7366x; 1.0434x over previous
import jax
import jax.numpy as jnp
from jax.experimental import pallas as pl


def _body(x_ref, p_ref, o_ref):
    o_ref[...] = x_ref[...] + p_ref[...]


def kernel(x, pos_table):
    B, S, D = x.shape
    BLK = 2048
    return pl.pallas_call(
        _body,
        grid=(S // BLK, B),
        in_specs=[
            pl.BlockSpec((1, BLK, D), lambda s, b: (b, s, 0)),
            pl.BlockSpec((BLK, D), lambda s, b: (s, 0)),
        ],
        out_specs=pl.BlockSpec((1, BLK, D), lambda s, b: (b, s, 0)),
        out_shape=jax.ShapeDtypeStruct((B, S, D), x.dtype),
    )(x, pos_table)
